# Initial kernel scaffold; baseline (speedup 1.0000x reference)
import jax
import jax.numpy as jnp
from jax.experimental import pallas as pl


def kernel(x, weight):
    raise NotImplementedError("write your pallas kernel here")


# speedup vs baseline: 1.0500x; 1.0500x over previous
import numpy as np
import jax
import jax.numpy as jnp
from jax import lax
from jax.experimental import pallas as pl
from jax.experimental.pallas import tpu as pltpu


def _make_kernel(H, W, HW, K, pad, strip, n_strips, n_per, n_rows):
    def _body(x_ref, w_ref, rc_ref, o_ref, acc_ref):
        j = pl.program_id(1)

        first = x_ref[0, 0:strip, :].astype(jnp.float32)
        s, m = first, first
        for k in range(1, n_strips):
            xs = x_ref[0, k * strip:(k + 1) * strip, :].astype(jnp.float32)
            s = s + xs
            m = jnp.maximum(m, xs)
        acc_ref[pl.ds(j, 1), :] = jnp.sum(s, axis=0, keepdims=True)
        acc_ref[pl.ds(n_per + j, 1), :] = jnp.max(m, axis=0, keepdims=True)

        @pl.when(j == n_per - 1)
        def _epilogue():
            two = acc_ref[...]
            row = rc_ref[0:1, :]
            col = rc_ref[1:2, :]

            s_dx = [jnp.zeros((n_rows, HW), jnp.float32) for _ in range(K)]
            for dy in range(K):
                shift = (-(dy - pad) * W) % HW
                r = two if shift == 0 else pltpu.roll(two, shift=shift, axis=1)
                rm = jnp.where((row >= pad - dy) & (row < H + pad - dy), r, 0.0)
                for dx in range(K):
                    w2 = w_ref[:, dy * K + dx:dy * K + dx + 1]
                    s_dx[dx] = s_dx[dx] + w2 * rm

            acc = jnp.zeros((n_rows, HW), jnp.float32)
            for dx in range(K):
                shift = (-(dx - pad)) % HW
                t = s_dx[dx] if shift == 0 else pltpu.roll(s_dx[dx],
                                                           shift=shift, axis=1)
                acc = acc + jnp.where((col >= pad - dx) & (col < W + pad - dx),
                                      t, 0.0)
            conv = acc[0:n_per, :] + acc[n_per:2 * n_per, :]
            o_ref[0] = jax.nn.sigmoid(conv).astype(o_ref.dtype)

    return _body


def kernel(x, weight):
    N, C, H, W = x.shape
    K = weight.shape[-1]
    pad = K // 2
    HW = H * W

    n_cores = 2 if N % 2 == 0 else 1
    n_per = N // n_cores
    n_rows = 2 * n_per
    strip = 8 if C % 8 == 0 else C
    n_strips = C // strip

    wt = weight.astype(jnp.float32).reshape(2, K * K)
    w_rows = jnp.concatenate(
        [jnp.broadcast_to(wt[0:1] / C, (n_per, K * K)),
         jnp.broadcast_to(wt[1:2], (n_per, K * K))], axis=0)

    rows = np.repeat(np.arange(H, dtype=np.int32), W)
    cols = np.tile(np.arange(W, dtype=np.int32), H)
    rc = jnp.asarray(np.stack([rows, cols], axis=0))

    xf = x.reshape(N, C, HW)

    kernel_fn = _make_kernel(H, W, HW, K, pad, strip, n_strips, n_per, n_rows)
    out = pl.pallas_call(
        kernel_fn,
        out_shape=jax.ShapeDtypeStruct((n_cores, n_per, HW), x.dtype),
        grid=(n_cores, n_per),
        in_specs=[
            pl.BlockSpec((1, C, HW), lambda i, j: (i * n_per + j, 0, 0)),
            pl.BlockSpec((n_rows, K * K), lambda i, j: (0, 0)),
            pl.BlockSpec((2, HW), lambda i, j: (0, 0)),
        ],
        out_specs=pl.BlockSpec((1, n_per, HW), lambda i, j: (i, 0, 0)),
        scratch_shapes=[pltpu.VMEM((n_rows, HW), jnp.float32)],
        compiler_params=pltpu.CompilerParams(
            dimension_semantics=("parallel", "arbitrary"),
            vmem_limit_bytes=48 * 1024 * 1024),
    )(xf, w_rows, rc)
    return out.reshape(N, 1, H, W)
